# baseline (speedup 1.0000x reference)
import functools

import jax
import jax.numpy as jnp
from jax import lax
from jax.experimental import pallas as pl
from jax.experimental.pallas import tpu as pltpu
from jax.experimental.pallas import tpu_sc as plsc

_L = 16
_QV = 4
_QG = _L * _QV


def _sc_body(x0h, x1h, x2h, w0h, w1h, w2h, wyh, disth, idxh,
             x0v, x1v, x2v, y0v, y1v, y2v, yyv, odv, oiv):
    info = plsc.get_sparse_core_info()
    nc = info.num_cores
    wid = lax.axis_index("s") * nc + lax.axis_index("c")
    n_w = x0v.shape[0]
    m = y0v.shape[0]
    base = wid * n_w
    batch = lax.div(base, m)
    pltpu.sync_copy(x0h.at[pl.ds(base, n_w)], x0v)
    pltpu.sync_copy(x1h.at[pl.ds(base, n_w)], x1v)
    pltpu.sync_copy(x2h.at[pl.ds(base, n_w)], x2v)
    pltpu.sync_copy(w0h.at[batch], y0v)
    pltpu.sync_copy(w1h.at[batch], y1v)
    pltpu.sync_copy(w2h.at[batch], y2v)
    pltpu.sync_copy(wyh.at[batch], yyv)

    inf = jnp.full((_L,), 3.0e38, jnp.float32)
    zero_i = jnp.zeros((_L,), jnp.int32)

    def per_group(g, _):
        qb = g * _QG
        xs = []
        for r in range(_QV):
            o = qb + r * _L
            xs.append((x0v[pl.ds(o, _L)], x1v[pl.ds(o, _L)], x2v[pl.ds(o, _L)]))

        def per_chunk(mc, carry):
            bv = list(carry[:_QV])
            bi = list(carry[_QV:])
            for j in range(_L):
                mi = mc * _L + j
                miv = jnp.full((_L,), mi, jnp.int32)
                y0 = plsc.load_gather(y0v, [miv])
                y1 = plsc.load_gather(y1v, [miv])
                y2 = plsc.load_gather(y2v, [miv])
                yy = plsc.load_gather(yyv, [miv])
                for r in range(_QV):
                    x0, x1, x2 = xs[r]
                    t = yy + x0 * y0 + x1 * y1 + x2 * y2
                    mask = t < bv[r]
                    bv[r] = jnp.where(mask, t, bv[r])
                    bi[r] = jnp.where(mask, miv, bi[r])
            return tuple(bv) + tuple(bi)

        carry = lax.fori_loop(0, m // _L, per_chunk,
                              (inf,) * _QV + (zero_i,) * _QV)
        for r in range(_QV):
            x0, x1, x2 = xs[r]
            xx = x0 * x0 + x1 * x1 + x2 * x2
            o = qb + r * _L
            odv[pl.ds(o, _L)] = carry[r] + xx
            oiv[pl.ds(o, _L)] = carry[_QV + r]
        return 0

    lax.fori_loop(0, n_w // _QG, per_group, 0)
    pltpu.sync_copy(odv, disth.at[pl.ds(base, n_w)])
    pltpu.sync_copy(oiv, idxh.at[pl.ds(base, n_w)])


def _sc_chamfer(x0, x1, x2, w0, w1, w2, wy):
    nq = x0.shape[0]
    b, m = w0.shape
    info = plsc.get_sparse_core_info()
    nw = info.num_cores * info.num_subcores
    n_w = nq // nw
    mesh = plsc.VectorSubcoreMesh(core_axis_name="c", subcore_axis_name="s")
    f = pl.kernel(
        _sc_body,
        out_type=[
            jax.ShapeDtypeStruct((nq,), jnp.float32),
            jax.ShapeDtypeStruct((nq,), jnp.int32),
        ],
        mesh=mesh,
        compiler_params=pltpu.CompilerParams(needs_layout_passes=False),
        scratch_types=[
            pltpu.VMEM((n_w,), jnp.float32),
            pltpu.VMEM((n_w,), jnp.float32),
            pltpu.VMEM((n_w,), jnp.float32),
            pltpu.VMEM((m,), jnp.float32),
            pltpu.VMEM((m,), jnp.float32),
            pltpu.VMEM((m,), jnp.float32),
            pltpu.VMEM((m,), jnp.float32),
            pltpu.VMEM((n_w,), jnp.float32),
            pltpu.VMEM((n_w,), jnp.int32),
        ],
    )
    return f(x0, x1, x2, w0, w1, w2, wy)


def kernel(input1, input2):
    b, n, _ = input1.shape
    m = input2.shape[1]
    x0 = input1[:, :, 0].reshape(-1)
    x1 = input1[:, :, 1].reshape(-1)
    x2 = input1[:, :, 2].reshape(-1)
    w0 = -2.0 * input2[:, :, 0]
    w1 = -2.0 * input2[:, :, 1]
    w2 = -2.0 * input2[:, :, 2]
    wy = jnp.sum(input2 * input2, axis=2)
    dist, idx = _sc_chamfer(x0, x1, x2, w0, w1, w2, wy)
    return dist.reshape(b, n), idx.reshape(b, n)


# speedup vs baseline: 2.5613x; 2.5613x over previous
import jax
import jax.numpy as jnp
from jax import lax
from jax.experimental import pallas as pl
from jax.experimental.pallas import tpu as pltpu
from jax.experimental.pallas import tpu_sc as plsc

_L = 16
_QV = 4
_QG = _L * _QV
_SC_FRAC = 0.3125


def _tc_body(x_ref, ya_ref, dist_ref, idx_ref):
    x = x_ref[0]
    ya = ya_ref[0]
    nt = x.shape[0]
    m = ya.shape[1]
    t = (x[:, 0:1] * ya[0:1, :] + ya[3:4, :]
         + x[:, 1:2] * ya[1:2, :]
         + x[:, 2:3] * ya[2:3, :])
    mn = jnp.min(t, axis=1, keepdims=True)
    iota = jax.lax.broadcasted_iota(jnp.int32, (nt, m), 1).astype(jnp.float32)
    idx_f = jnp.min(jnp.where(t == mn, iota, jnp.float32(m)), axis=1)
    c0 = x[:, 0:1]
    c1 = x[:, 1:2]
    c2 = x[:, 2:3]
    xx = c0 * c0 + c1 * c1 + c2 * c2
    dist_ref[0, 0, 0] = (mn + xx)[:, 0]
    idx_ref[0, 0, 0] = idx_f.astype(jnp.int32)


def _tc_chamfer(x, ya):
    b, n, _ = x.shape
    m = ya.shape[2]
    nt = n
    for cand in (512, 256, 128, 64, 32, 16, 8):
        if n % cand == 0:
            nt = cand
            break
    n_tiles = n // nt
    dist, idx = pl.pallas_call(
        _tc_body,
        grid=(b, n_tiles),
        in_specs=[
            pl.BlockSpec((1, nt, 3), lambda bi, i: (bi, i, 0)),
            pl.BlockSpec((1, 4, m), lambda bi, i: (bi, 0, 0)),
        ],
        out_specs=[
            pl.BlockSpec((1, 1, 1, nt), lambda bi, i: (bi, i, 0, 0)),
            pl.BlockSpec((1, 1, 1, nt), lambda bi, i: (bi, i, 0, 0)),
        ],
        out_shape=[
            jax.ShapeDtypeStruct((b, n_tiles, 1, nt), jnp.float32),
            jax.ShapeDtypeStruct((b, n_tiles, 1, nt), jnp.int32),
        ],
    )(x, ya)
    return dist.reshape(b, n), idx.reshape(b, n)


def _sc_body(x0h, x1h, x2h, w0h, w1h, w2h, wyh, disth, idxh,
             x0v, x1v, x2v, y0v, y1v, y2v, yyv, odv, oiv):
    info = plsc.get_sparse_core_info()
    nc = info.num_cores
    wid = lax.axis_index("s") * nc + lax.axis_index("c")
    n_w = x0v.shape[0]
    m = y0v.shape[0]
    n_per_batch = x0h.shape[0] // w0h.shape[0]
    base = wid * n_w
    batch = lax.div(base, n_per_batch)
    pltpu.sync_copy(x0h.at[pl.ds(base, n_w)], x0v)
    pltpu.sync_copy(x1h.at[pl.ds(base, n_w)], x1v)
    pltpu.sync_copy(x2h.at[pl.ds(base, n_w)], x2v)
    pltpu.sync_copy(w0h.at[batch], y0v)
    pltpu.sync_copy(w1h.at[batch], y1v)
    pltpu.sync_copy(w2h.at[batch], y2v)
    pltpu.sync_copy(wyh.at[batch], yyv)

    inf = jnp.full((_L,), 3.0e38, jnp.float32)
    zero_i = jnp.zeros((_L,), jnp.int32)

    def per_group(g, _):
        qb = g * _QG
        xs = []
        for r in range(_QV):
            o = qb + r * _L
            xs.append((x0v[pl.ds(o, _L)], x1v[pl.ds(o, _L)], x2v[pl.ds(o, _L)]))

        def per_chunk(mc, carry):
            bv = list(carry[:_QV])
            bi = list(carry[_QV:])
            for j in range(_L):
                mi = mc * _L + j
                miv = jnp.full((_L,), mi, jnp.int32)
                y0 = plsc.load_gather(y0v, [miv])
                y1 = plsc.load_gather(y1v, [miv])
                y2 = plsc.load_gather(y2v, [miv])
                yy = plsc.load_gather(yyv, [miv])
                for r in range(_QV):
                    x0, x1, x2 = xs[r]
                    t = yy + x0 * y0 + x1 * y1 + x2 * y2
                    mask = t < bv[r]
                    bv[r] = jnp.where(mask, t, bv[r])
                    bi[r] = jnp.where(mask, miv, bi[r])
            return tuple(bv) + tuple(bi)

        carry = lax.fori_loop(0, m // _L, per_chunk,
                              (inf,) * _QV + (zero_i,) * _QV)
        for r in range(_QV):
            x0, x1, x2 = xs[r]
            xx = x0 * x0 + x1 * x1 + x2 * x2
            o = qb + r * _L
            odv[pl.ds(o, _L)] = carry[r] + xx
            oiv[pl.ds(o, _L)] = carry[_QV + r]
        return 0

    lax.fori_loop(0, n_w // _QG, per_group, 0)
    pltpu.sync_copy(odv, disth.at[pl.ds(base, n_w)])
    pltpu.sync_copy(oiv, idxh.at[pl.ds(base, n_w)])


def _sc_chamfer(x0, x1, x2, w0, w1, w2, wy):
    nq = x0.shape[0]
    m = w0.shape[1]
    info = plsc.get_sparse_core_info()
    nw = info.num_cores * info.num_subcores
    n_w = nq // nw
    mesh = plsc.VectorSubcoreMesh(core_axis_name="c", subcore_axis_name="s")
    f = pl.kernel(
        _sc_body,
        out_type=[
            jax.ShapeDtypeStruct((nq,), jnp.float32),
            jax.ShapeDtypeStruct((nq,), jnp.int32),
        ],
        mesh=mesh,
        compiler_params=pltpu.CompilerParams(needs_layout_passes=False),
        scratch_types=[
            pltpu.VMEM((n_w,), jnp.float32),
            pltpu.VMEM((n_w,), jnp.float32),
            pltpu.VMEM((n_w,), jnp.float32),
            pltpu.VMEM((m,), jnp.float32),
            pltpu.VMEM((m,), jnp.float32),
            pltpu.VMEM((m,), jnp.float32),
            pltpu.VMEM((m,), jnp.float32),
            pltpu.VMEM((n_w,), jnp.float32),
            pltpu.VMEM((n_w,), jnp.int32),
        ],
    )
    return f(x0, x1, x2, w0, w1, w2, wy)


def _sc_split(b, n):
    if b != 8:
        return 0
    ns = (int(n * _SC_FRAC) // 256) * 256
    if ns <= 0 or ns >= n:
        return 0
    return ns


def kernel(input1, input2):
    b, n, _ = input1.shape
    m = input2.shape[1]
    yt = jnp.transpose(input2, (0, 2, 1))
    wy = jnp.sum(input2 * input2, axis=2)
    ya = jnp.concatenate([-2.0 * yt, wy[:, None, :]], axis=1)
    ns = _sc_split(b, n)
    if ns == 0:
        dist, idx = _tc_chamfer(input1, ya)
        return dist, idx
    n_tc = n - ns
    dist_tc, idx_tc = _tc_chamfer(input1[:, :n_tc], ya)
    xs = input1[:, n_tc:]
    sc_dist, sc_idx = _sc_chamfer(
        xs[:, :, 0].reshape(-1), xs[:, :, 1].reshape(-1),
        xs[:, :, 2].reshape(-1),
        -2.0 * input2[:, :, 0], -2.0 * input2[:, :, 1],
        -2.0 * input2[:, :, 2], wy)
    dist = jnp.concatenate([dist_tc, sc_dist.reshape(b, ns)], axis=1)
    idx = jnp.concatenate([idx_tc, sc_idx.reshape(b, ns)], axis=1)
    return dist, idx
